# baseline (speedup 1.0000x reference)
import functools

import jax
import jax.numpy as jnp
from jax import lax
from jax.experimental import pallas as pl
from jax.experimental.pallas import tpu as pltpu
from jax.experimental.pallas import tpu_sc as plsc

NC = 2
NS = 16
NW = NC * NS
L = 16
CH = 128
NBUF = 5
CU = 4


@functools.partial(jax.jit, static_argnums=(2, 3))
def _sc_lookup(ids, table_flat, B, D):
    b_per_w = B // NW
    iters = b_per_w // CH
    rounds = iters // NBUF
    ng = CH // L
    mesh = plsc.VectorSubcoreMesh(core_axis_name="c", subcore_axis_name="s")

    @functools.partial(
        pl.kernel,
        mesh=mesh,
        compiler_params=pltpu.CompilerParams(needs_layout_passes=False),
        out_type=jax.ShapeDtypeStruct((B * D,), jnp.float32),
        scratch_types=[
            pltpu.VMEM((b_per_w,), jnp.int32),
            pltpu.VMEM((5 * D,), jnp.float32),
            [pltpu.VMEM((CH * D,), jnp.float32) for _ in range(NBUF)],
            pltpu.SemaphoreType.DMA,
            [pltpu.SemaphoreType.DMA for _ in range(NBUF)],
        ],
    )
    def k(ids_hbm, table_hbm, out_hbm, idx_v, table_v, rows, gsem, wsems):
        wid = lax.axis_index("s") * NC + lax.axis_index("c")
        base = wid * b_per_w

        pltpu.sync_copy(table_hbm, table_v)
        pltpu.async_copy(ids_hbm.at[pl.ds(base, b_per_w)], idx_v, gsem).wait()

        lane = lax.iota(jnp.int32, L)
        pos128 = [(g * L + lane) * D for g in range(ng)]

        def build(t, rows_b):
            ids128 = [
                idx_v[pl.ds(t * CH + g * L, L)] * D for g in range(ng)
            ]

            @plsc.parallel_loop(0, D, unroll=CU)
            def _cols(c):
                for g in range(ng):
                    vals = plsc.load_gather(table_v, [ids128[g] + c])
                    plsc.store_scatter(rows_b, [pos128[g] + c], vals)

        def write(t, b):
            return pltpu.make_async_copy(
                rows[b], out_hbm.at[pl.ds((base + t * CH) * D, CH * D)], wsems[b]
            )

        def round_body(r, carry):
            t0 = r * NBUF
            for b in range(NBUF):
                t = t0 + b

                @pl.when(r > 0)
                def _():
                    write(t - NBUF, b).wait()

                build(t, rows[b])
                write(t, b).start()
            return carry

        lax.fori_loop(0, rounds, round_body, 0)

        for b in range(NBUF):
            write(iters - NBUF + b, b).wait()

    return k(ids, table_flat)


def kernel(modality_ids, embedding_table):
    Bb, S = modality_ids.shape
    V, D = embedding_table.shape
    B = Bb * S
    ids = modality_ids.reshape(B).astype(jnp.int32)
    out = _sc_lookup(ids, embedding_table.reshape(V * D), B, D)
    return out.reshape(Bb, S, D)


# speedup vs baseline: 1.0190x; 1.0190x over previous
import functools

import jax
import jax.numpy as jnp
from jax import lax
from jax.experimental import pallas as pl
from jax.experimental.pallas import tpu as pltpu
from jax.experimental.pallas import tpu_sc as plsc

NC = 2
NS = 16
NW = NC * NS
L = 16
CH = 128
NBUF = 5
CU = 8


@functools.partial(jax.jit, static_argnums=(2, 3))
def _sc_lookup(ids, table_flat, B, D):
    b_per_w = B // NW
    iters = b_per_w // CH
    rounds = iters // NBUF
    ng = CH // L
    mesh = plsc.VectorSubcoreMesh(core_axis_name="c", subcore_axis_name="s")

    @functools.partial(
        pl.kernel,
        mesh=mesh,
        compiler_params=pltpu.CompilerParams(needs_layout_passes=False),
        out_type=jax.ShapeDtypeStruct((B * D,), jnp.float32),
        scratch_types=[
            pltpu.VMEM((b_per_w,), jnp.int32),
            pltpu.VMEM((5 * D,), jnp.float32),
            [pltpu.VMEM((CH * D,), jnp.float32) for _ in range(NBUF)],
            pltpu.SemaphoreType.DMA,
            [pltpu.SemaphoreType.DMA for _ in range(NBUF)],
        ],
    )
    def k(ids_hbm, table_hbm, out_hbm, idx_v, table_v, rows, gsem, wsems):
        wid = lax.axis_index("s") * NC + lax.axis_index("c")
        base = wid * b_per_w

        pltpu.sync_copy(table_hbm, table_v)
        pltpu.async_copy(ids_hbm.at[pl.ds(base, b_per_w)], idx_v, gsem).wait()

        lane = lax.iota(jnp.int32, L)
        pos128 = [(g * L + lane) * D for g in range(ng)]

        def build(t, rows_b):
            ids128 = [
                idx_v[pl.ds(t * CH + g * L, L)] * D for g in range(ng)
            ]

            for g in range(ng):
                @plsc.parallel_loop(0, D, unroll=CU)
                def _cols(c, _ids=ids128[g], _pos=pos128[g]):
                    vals = plsc.load_gather(table_v, [_ids + c])
                    plsc.store_scatter(rows_b, [_pos + c], vals)

        def write(t, b):
            return pltpu.make_async_copy(
                rows[b], out_hbm.at[pl.ds((base + t * CH) * D, CH * D)], wsems[b]
            )

        def round_body(r, carry):
            t0 = r * NBUF
            for b in range(NBUF):
                t = t0 + b

                @pl.when(r > 0)
                def _():
                    write(t - NBUF, b).wait()

                build(t, rows[b])
                write(t, b).start()
            return carry

        lax.fori_loop(0, rounds, round_body, 0)

        for b in range(NBUF):
            write(iters - NBUF + b, b).wait()

    return k(ids, table_flat)


def kernel(modality_ids, embedding_table):
    Bb, S = modality_ids.shape
    V, D = embedding_table.shape
    B = Bb * S
    ids = modality_ids.reshape(B).astype(jnp.int32)
    out = _sc_lookup(ids, embedding_table.reshape(V * D), B, D)
    return out.reshape(Bb, S, D)


# speedup vs baseline: 6.5878x; 6.4650x over previous
import functools

import jax
import jax.numpy as jnp
from jax import lax
from jax.experimental import pallas as pl
from jax.experimental.pallas import tpu as pltpu
from jax.experimental.pallas import tpu_sc as plsc

NC = 2
NS = 16
NW = NC * NS
L = 16
CH = 128
NBUF = 5
CU = 8


@functools.partial(jax.jit, static_argnums=(2, 3))
def _sc_lookup(ids, table_flat, B, D):
    b_per_w = B // NW
    iters = b_per_w // CH
    rounds = iters // NBUF
    ng = CH // L
    mesh = plsc.VectorSubcoreMesh(core_axis_name="c", subcore_axis_name="s")

    @functools.partial(
        pl.kernel,
        mesh=mesh,
        compiler_params=pltpu.CompilerParams(needs_layout_passes=False),
        out_type=jax.ShapeDtypeStruct((B * D,), jnp.float32),
        scratch_types=[
            pltpu.VMEM((b_per_w,), jnp.int32),
            pltpu.VMEM((5 * D,), jnp.float32),
            [pltpu.VMEM((CH * D,), jnp.float32) for _ in range(NBUF)],
            pltpu.SemaphoreType.DMA,
            [pltpu.SemaphoreType.DMA for _ in range(NBUF)],
        ],
    )
    def k(ids_hbm, table_hbm, out_hbm, idx_v, table_v, rows, gsem, wsems):
        wid = lax.axis_index("s") * NC + lax.axis_index("c")
        base = wid * b_per_w

        pltpu.sync_copy(table_hbm, table_v)
        pltpu.async_copy(ids_hbm.at[pl.ds(base, b_per_w)], idx_v, gsem).wait()

        lane = lax.iota(jnp.int32, L)
        pos128 = [(g * L + lane) * D for g in range(ng)]

        def build(t, rows_b):
            ids128 = [
                idx_v[pl.ds(t * CH + g * L, L)] * D for g in range(ng)
            ]

            @plsc.parallel_loop(0, D, unroll=CU)
            def _cols(c):
                diag = (lane + c) & (D - 1)
                for g in range(ng):
                    vals = plsc.load_gather(table_v, [ids128[g] + diag])
                    plsc.store_scatter(rows_b, [pos128[g] + diag], vals)

        def write(t, b):
            return pltpu.make_async_copy(
                rows[b], out_hbm.at[pl.ds((base + t * CH) * D, CH * D)], wsems[b]
            )

        def round_body(r, carry):
            t0 = r * NBUF
            for b in range(NBUF):
                t = t0 + b

                @pl.when(r > 0)
                def _():
                    write(t - NBUF, b).wait()

                build(t, rows[b])
                write(t, b).start()
            return carry

        lax.fori_loop(0, rounds, round_body, 0)

        for b in range(NBUF):
            write(iters - NBUF + b, b).wait()

    return k(ids, table_flat)


def kernel(modality_ids, embedding_table):
    Bb, S = modality_ids.shape
    V, D = embedding_table.shape
    B = Bb * S
    ids = modality_ids.reshape(B).astype(jnp.int32)
    out = _sc_lookup(ids, embedding_table.reshape(V * D), B, D)
    return out.reshape(Bb, S, D)


# speedup vs baseline: 6.5899x; 1.0003x over previous
import functools

import jax
import jax.numpy as jnp
from jax import lax
from jax.experimental import pallas as pl
from jax.experimental.pallas import tpu as pltpu
from jax.experimental.pallas import tpu_sc as plsc

NC = 2
NS = 16
NW = NC * NS
L = 16
CH = 128
NBUF = 5
CU = 16


@functools.partial(jax.jit, static_argnums=(2, 3))
def _sc_lookup(ids, table_flat, B, D):
    b_per_w = B // NW
    iters = b_per_w // CH
    rounds = iters // NBUF
    ng = CH // L
    mesh = plsc.VectorSubcoreMesh(core_axis_name="c", subcore_axis_name="s")

    @functools.partial(
        pl.kernel,
        mesh=mesh,
        compiler_params=pltpu.CompilerParams(needs_layout_passes=False),
        out_type=jax.ShapeDtypeStruct((B * D,), jnp.float32),
        scratch_types=[
            pltpu.VMEM((b_per_w,), jnp.int32),
            pltpu.VMEM((5 * D,), jnp.float32),
            [pltpu.VMEM((CH * D,), jnp.float32) for _ in range(NBUF)],
            pltpu.SemaphoreType.DMA,
            [pltpu.SemaphoreType.DMA for _ in range(NBUF)],
        ],
    )
    def k(ids_hbm, table_hbm, out_hbm, idx_v, table_v, rows, gsem, wsems):
        wid = lax.axis_index("s") * NC + lax.axis_index("c")
        base = wid * b_per_w

        pltpu.sync_copy(table_hbm, table_v)
        pltpu.async_copy(ids_hbm.at[pl.ds(base, b_per_w)], idx_v, gsem).wait()

        lane = lax.iota(jnp.int32, L)
        pos128 = [(g * L + lane) * D for g in range(ng)]

        def build(t, rows_b):
            ids128 = [
                idx_v[pl.ds(t * CH + g * L, L)] * D for g in range(ng)
            ]

            @plsc.parallel_loop(0, D, unroll=CU)
            def _cols(c):
                diag = (lane + c) & (D - 1)
                for g in range(ng):
                    vals = plsc.load_gather(table_v, [ids128[g] + diag])
                    plsc.store_scatter(rows_b, [pos128[g] + diag], vals)

        def write(t, b):
            return pltpu.make_async_copy(
                rows[b], out_hbm.at[pl.ds((base + t * CH) * D, CH * D)], wsems[b]
            )

        def round_body(r, carry):
            t0 = r * NBUF
            for b in range(NBUF):
                t = t0 + b

                @pl.when(r > 0)
                def _():
                    write(t - NBUF, b).wait()

                build(t, rows[b])
                write(t, b).start()
            return carry

        lax.fori_loop(0, rounds, round_body, 0)

        for b in range(NBUF):
            write(iters - NBUF + b, b).wait()

    return k(ids, table_flat)


def kernel(modality_ids, embedding_table):
    Bb, S = modality_ids.shape
    V, D = embedding_table.shape
    B = Bb * S
    ids = modality_ids.reshape(B).astype(jnp.int32)
    out = _sc_lookup(ids, embedding_table.reshape(V * D), B, D)
    return out.reshape(Bb, S, D)
